# baseline (speedup 1.0000x reference)
import functools
import math

import jax
import jax.numpy as jnp
from jax import lax
from jax.experimental import pallas as pl
from jax.experimental.pallas import tpu as pltpu
from jax.experimental.pallas import tpu_sc as plsc

VOCAB = 1000000
D = 64
SCALE = math.sqrt(D)

_info = plsc.get_sparse_core_info()
NC, NS, L = _info.num_cores, _info.num_subcores, _info.num_lanes
NW = NC * NS

CHUNK = 128
B_TOTAL = 4096 * 200
B_PER_W = B_TOTAL // NW
N_CHUNKS = B_PER_W // CHUNK
NBUF = 4
N_GROUPS = N_CHUNKS // NBUF
UNROLL = 4


def _sc_gather(idx_hbm, table_hbm, out_hbm, *scratch):
    idx_v = scratch[0]
    gbufs = scratch[1:1 + NBUF]
    sbufs = scratch[1 + NBUF:1 + 2 * NBUF]
    gsems = scratch[1 + 2 * NBUF:1 + 3 * NBUF]
    ssems = scratch[1 + 3 * NBUF:1 + 4 * NBUF]

    wid = lax.axis_index("s") * NC + lax.axis_index("c")
    base = wid * B_PER_W

    pltpu.sync_copy(idx_hbm.at[wid], idx_v)

    def gather_start(j, b):
        pltpu.async_copy(table_hbm.at[idx_v.at[j]], gbufs[b], gsems[b])

    def gather_wait(b):
        pltpu.make_async_copy(table_hbm.at[idx_v.at[0]], gbufs[b],
                              gsems[b]).wait()

    def store_start(j, b):
        pltpu.async_copy(sbufs[b], out_hbm.at[pl.ds(base + j * CHUNK, CHUNK)],
                         ssems[b])

    def store_wait(b):
        pltpu.make_async_copy(sbufs[b], out_hbm.at[pl.ds(base, CHUNK)],
                              ssems[b]).wait()

    def scale(b):
        g, s = gbufs[b], sbufs[b]

        def scale_rows(r0, _):
            for u in range(UNROLL):
                for c in range(D // L):
                    sl = pl.ds(c * L, L)
                    s[r0 + u, sl] = g[r0 + u, sl] * SCALE
            return 0

        lax.fori_loop(0, CHUNK // UNROLL, lambda i, _: scale_rows(i * UNROLL, _),
                      0, unroll=False)

    for b in range(NBUF):
        gather_start(b, b)

    def group_body(g, _):
        for b in range(NBUF):
            j = g * NBUF + b
            gather_wait(b)

            @pl.when(g > 0)
            def _():
                store_wait(b)

            scale(b)
            store_start(j, b)

            @pl.when(g < N_GROUPS - 1)
            def _():
                gather_start(j + NBUF, b)
        return 0

    lax.fori_loop(0, N_GROUPS, group_body, 0)
    for b in range(NBUF):
        store_wait(b)


def _tc_transpose(x_ref, o_ref):
    o_ref[0] = jnp.transpose(x_ref[0, 0], (1, 0))


@jax.jit
def kernel(x, table):
    idx = x.reshape(NW, N_CHUNKS, CHUNK)
    mesh = plsc.VectorSubcoreMesh(core_axis_name="c", subcore_axis_name="s")
    rows = pl.kernel(
        _sc_gather,
        mesh=mesh,
        compiler_params=pltpu.CompilerParams(use_tc_tiling_on_sc=False),
        out_type=jax.ShapeDtypeStruct((B_TOTAL, D), jnp.float32),
        scratch_types=(
            [pltpu.VMEM((N_CHUNKS, CHUNK), jnp.int32)]
            + [pltpu.VMEM((CHUNK, D), jnp.float32) for _ in range(2 * NBUF)]
            + [pltpu.SemaphoreType.DMA for _ in range(2 * NBUF)]
        ),
    )(idx, table)

    rows4 = rows.reshape(200, 32, CHUNK, D)
    out_t = pl.pallas_call(
        _tc_transpose,
        grid=(200, 32),
        in_specs=[pl.BlockSpec((1, 1, CHUNK, D), lambda s, c: (s, c, 0, 0))],
        out_specs=pl.BlockSpec((1, D, CHUNK), lambda s, c: (s, 0, c)),
        out_shape=jax.ShapeDtypeStruct((200, D, 4096), jnp.float32),
    )(rows4)

    return out_t.transpose(2, 0, 1)


# speedup vs baseline: 3.5824x; 3.5824x over previous
import functools
import math

import jax
import jax.numpy as jnp
from jax import lax
from jax.experimental import pallas as pl
from jax.experimental.pallas import tpu as pltpu
from jax.experimental.pallas import tpu_sc as plsc

VOCAB = 1000000
D = 64
SCALE = math.sqrt(D)

_info = plsc.get_sparse_core_info()
NC, NS, L = _info.num_cores, _info.num_subcores, _info.num_lanes
NW = NC * NS

CHUNK = 128
B_TOTAL = 4096 * 200
B_PER_W = B_TOTAL // NW
N_CHUNKS = B_PER_W // CHUNK
NBUF = 4
N_GROUPS = N_CHUNKS // NBUF
UNROLL = 4


def _sc_gather(idx_hbm, table_hbm, out_hbm, *scratch):
    idx_v = scratch[0]
    gbufs = scratch[1:1 + NBUF]
    sbufs = scratch[1 + NBUF:1 + 2 * NBUF]
    gsems = scratch[1 + 2 * NBUF:1 + 3 * NBUF]
    ssems = scratch[1 + 3 * NBUF:1 + 4 * NBUF]

    wid = lax.axis_index("s") * NC + lax.axis_index("c")
    base = wid * B_PER_W

    pltpu.sync_copy(idx_hbm.at[wid], idx_v)

    def gather_start(j, b):
        pltpu.async_copy(table_hbm.at[idx_v.at[j]], gbufs[b], gsems[b])

    def gather_wait(b):
        pltpu.make_async_copy(table_hbm.at[idx_v.at[0]], gbufs[b],
                              gsems[b]).wait()

    def store_start(j, b):
        pltpu.async_copy(sbufs[b], out_hbm.at[pl.ds(base + j * CHUNK, CHUNK)],
                         ssems[b])

    def store_wait(b):
        pltpu.make_async_copy(sbufs[b], out_hbm.at[pl.ds(base, CHUNK)],
                              ssems[b]).wait()

    def scale(b):
        g, s = gbufs[b], sbufs[b]

        def scale_rows(r0, _):
            for u in range(UNROLL):
                for c in range(D // L):
                    sl = pl.ds(c * L, L)
                    s[r0 + u, sl] = g[r0 + u, sl] * SCALE
            return 0

        lax.fori_loop(0, CHUNK // UNROLL, lambda i, _: scale_rows(i * UNROLL, _),
                      0, unroll=False)

    for b in range(NBUF):
        gather_start(b, b)

    def group_body(g, _):
        for b in range(NBUF):
            j = g * NBUF + b
            gather_wait(b)

            @pl.when(g > 0)
            def _():
                store_wait(b)

            scale(b)
            store_start(j, b)

            @pl.when(g < N_GROUPS - 1)
            def _():
                gather_start(j + NBUF, b)
        return 0

    lax.fori_loop(0, N_GROUPS, group_body, 0)
    for b in range(NBUF):
        store_wait(b)


@jax.jit
def kernel(x, table):
    idx = x.reshape(NW, N_CHUNKS, CHUNK)
    mesh = plsc.VectorSubcoreMesh(core_axis_name="c", subcore_axis_name="s")
    rows = pl.kernel(
        _sc_gather,
        mesh=mesh,
        compiler_params=pltpu.CompilerParams(use_tc_tiling_on_sc=False),
        out_type=jax.ShapeDtypeStruct((B_TOTAL, D), jnp.float32),
        scratch_types=(
            [pltpu.VMEM((N_CHUNKS, CHUNK), jnp.int32)]
            + [pltpu.VMEM((CHUNK, D), jnp.float32) for _ in range(2 * NBUF)]
            + [pltpu.SemaphoreType.DMA for _ in range(2 * NBUF)]
        ),
    )(idx, table)

    return rows.reshape(4096, 200, D)
